# baseline (speedup 1.0000x reference)
import functools

import numpy as np
import jax
import jax.numpy as jnp
from jax import lax
from jax.experimental import pallas as pl
from jax.experimental.pallas import tpu as pltpu
from jax.experimental.pallas import tpu_sc as plsc

_D = 256
_H = 8
_L = 4
_P = 4
_HD = _D // _H
_SPATIAL = np.array([[64, 64], [32, 32], [16, 16], [8, 8]], dtype=np.int64)
_LSI = np.concatenate([np.zeros(1, np.int64),
                       np.cumsum(_SPATIAL[:, 0] * _SPATIAL[:, 1])[:-1]])
_NK = int((_SPATIAL[:, 0] * _SPATIAL[:, 1]).sum())
_NQ = _NK
_NROWS = _NQ * _H

_j = np.arange(_H * _L * _P)
_l_of_j = (_j // _P) % _L
_h_of_j = _j // (_L * _P)
_WJ = _SPATIAL[_l_of_j, 1].astype(np.float32).reshape(1, 128)
_HJ = _SPATIAL[_l_of_j, 0].astype(np.float32).reshape(1, 128)
_WJI = _SPATIAL[_l_of_j, 1].astype(np.int32).reshape(1, 128)
_HJI = _SPATIAL[_l_of_j, 0].astype(np.int32).reshape(1, 128)
_BASEJ = (_LSI[_l_of_j] * _H + _h_of_j).astype(np.int32).reshape(1, 128)
_GS = np.kron(np.eye(_H, dtype=np.float32),
              np.ones((_L * _P, _L * _P), np.float32))

_PERM = np.zeros(_D, np.int64)
for _h in range(_H):
    for _i in range(16):
        _PERM[_h * 32 + 2 * _i] = _h * 32 + _i
        _PERM[_h * 32 + 2 * _i + 1] = _h * 32 + 16 + _i

_NWORK = 32
_RPW = _NROWS // _NWORK
_CHQ = 2
_CH = _CHQ * _H
_NCH = _RPW // _CH
_EPC = _CH * 64


def _stage_a_body(q_ref, rpx_ref, rpy_ref, x_ref, Wv_ref, bv_ref,
                  Wox_ref, Woy_ref, box_ref, boy_ref, Wa_ref, ba_ref,
                  gs_ref, wjf_ref, hjf_ref, wji_ref, hji_ref, basej_ref,
                  value_ref, sx_ref, sy_ref, aw_ref, idx_ref, w_ref):
    f32 = jnp.float32
    q = q_ref[...]
    value_ref[...] = (jnp.dot(x_ref[...], Wv_ref[...],
                              preferred_element_type=f32)
                      + bv_ref[...]).astype(jnp.bfloat16)
    offx = jnp.dot(q, Wox_ref[...], preferred_element_type=f32) + box_ref[...]
    offy = jnp.dot(q, Woy_ref[...], preferred_element_type=f32) + boy_ref[...]
    logits = jnp.dot(q, Wa_ref[...], preferred_element_type=f32) + ba_ref[...]
    e = jnp.exp(logits)
    aw = e / jnp.dot(e, gs_ref[...], preferred_element_type=f32)
    aw_ref[...] = aw

    wjf = wjf_ref[...]
    hjf = hjf_ref[...]
    lane = lax.broadcasted_iota(jnp.int32, (1, 128), 1)
    lane_l = (lane // _P) % _L
    rx = jnp.zeros_like(offx)
    ry = jnp.zeros_like(offy)
    for l in range(_L):
        m = (lane_l == l).astype(f32)
        rx = rx + rpx_ref[:, l:l + 1] * m
        ry = ry + rpy_ref[:, l:l + 1] * m
    locx = rx + offx / wjf
    locy = ry + offy / hjf
    sx_ref[...] = locx
    sy_ref[...] = locy

    x = locx * wjf - 0.5
    y = locy * hjf - 0.5
    x0 = jnp.floor(x)
    y0 = jnp.floor(y)
    lx = x - x0
    ly = y - y0
    xi = x0.astype(jnp.int32)
    yi = y0.astype(jnp.int32)
    wji = wji_ref[...]
    hji = hji_ref[...]
    basej = basej_ref[...]

    def corner(dx, dy, wgt):
        cx = xi + dx
        cy = yi + dy
        valid = (cx >= 0) & (cx < wji) & (cy >= 0) & (cy < hji)
        ccx = jnp.clip(cx, 0, wji - 1)
        ccy = jnp.clip(cy, 0, hji - 1)
        idx = basej + (ccy * wji + ccx) * _H
        w = jnp.where(valid, wgt, 0.0) * aw
        return idx, w

    bq = xi.shape[0]
    for c, (dx, dy, wgt) in enumerate((
            (0, 0, (1 - lx) * (1 - ly)), (1, 0, lx * (1 - ly)),
            (0, 1, (1 - lx) * ly), (1, 1, lx * ly))):
        idx, w = corner(dx, dy, wgt)
        idx_ref[:, c:c + 1, :] = idx.reshape(bq, 1, 128)
        w_ref[:, c:c + 1, :] = w.reshape(bq, 1, 128)


def _stage_a(q2, rpx, rpy, x2, Wv, bv2, Wox, Woy, box, boy, Wa, ba2):
    grid = 4
    bq = _NQ // grid
    f32 = jnp.float32
    i32 = jnp.int32
    row_spec = lambda w: pl.BlockSpec((bq, w), lambda i: (i, 0))
    full_spec = lambda a: pl.BlockSpec(a.shape, lambda i: (0,) * a.ndim)
    consts = [jnp.asarray(c) for c in
              (_GS, _WJ, _HJ, _WJI, _HJI, _BASEJ)]
    out_shapes = ([jax.ShapeDtypeStruct((_NQ, _D), jnp.bfloat16)]
                  + [jax.ShapeDtypeStruct((_NQ, 128), f32)] * 3
                  + [jax.ShapeDtypeStruct((_NQ, 4, 128), i32),
                     jax.ShapeDtypeStruct((_NQ, 4, 128), f32)])
    out_specs = ([row_spec(_D)] + [row_spec(128)] * 3
                 + [pl.BlockSpec((bq, 4, 128), lambda i: (i, 0, 0))] * 2)
    in_arrays = (q2, rpx, rpy, x2, Wv, bv2, Wox, Woy, box, boy, Wa, ba2,
                 *consts)
    in_specs = [row_spec(_D), row_spec(_L), row_spec(_L), row_spec(_D)] + \
               [full_spec(a) for a in in_arrays[4:]]
    return pl.pallas_call(
        _stage_a_body,
        grid=(grid,),
        in_specs=in_specs,
        out_specs=out_specs,
        out_shape=out_shapes,
    )(*in_arrays)


def _sc_combine(value_rows, idx_all, w_all):
    mesh = plsc.VectorSubcoreMesh(core_axis_name="c", subcore_axis_name="s")
    f32 = jnp.float32

    @functools.partial(
        pl.kernel,
        mesh=mesh,
        compiler_params=pltpu.CompilerParams(use_tc_tiling_on_sc=False),
        out_type=jax.ShapeDtypeStruct((_NROWS * _HD,), f32),
        scratch_types=[
            pltpu.VMEM((2, _EPC), jnp.int32),
            pltpu.VMEM((2, _EPC), f32),
            pltpu.VMEM((2, _EPC, 16), jnp.int32),
            pltpu.VMEM((2, _CH * _HD), f32),
            pltpu.SemaphoreType.DMA,
            pltpu.SemaphoreType.DMA,
            pltpu.SemaphoreType.DMA,
            pltpu.SemaphoreType.DMA,
            pltpu.SemaphoreType.DMA,
            pltpu.SemaphoreType.DMA,
        ],
    )
    def k(val_hbm, idx_hbm, w_hbm, out_hbm,
          idx_v, w_v, g_v, o_v,
          sem_in0, sem_in1, sem_g0, sem_g1, sem_o0, sem_o1):
        sem_in = (sem_in0, sem_in1)
        sem_g = (sem_g0, sem_g1)
        sem_o = (sem_o0, sem_o1)
        wid = lax.axis_index("s") * 2 + lax.axis_index("c")
        row0 = wid * _RPW

        def in_copies(ci, b):
            e0 = (row0 + ci * _CH) * 64
            return (pltpu.make_async_copy(idx_hbm.at[pl.ds(e0, _EPC)],
                                          idx_v.at[b], sem_in[b]),
                    pltpu.make_async_copy(w_hbm.at[pl.ds(e0, _EPC)],
                                          w_v.at[b], sem_in[b]))

        def gather(b):
            return pltpu.make_async_copy(val_hbm.at[idx_v.at[b]],
                                         g_v.at[b], sem_g[b])

        def out_copy(ci, b):
            return pltpu.make_async_copy(
                o_v.at[b],
                out_hbm.at[pl.ds((row0 + ci * _CH) * _HD, _CH * _HD)],
                sem_o[b])

        def start(copies):
            for cp in (copies if isinstance(copies, tuple) else (copies,)):
                cp.start()

        def wait(copies):
            for cp in (copies if isinstance(copies, tuple) else (copies,)):
                cp.wait()

        start(in_copies(0, 0))
        start(in_copies(1, 1))
        wait(in_copies(0, 0))
        start(gather(0))

        def step(i, b):
            wait(gather(b))

            @pl.when(i + 1 < _NCH)
            def _():
                wait(in_copies(i + 1, 1 - b))
                start(gather(1 - b))

            @pl.when(i >= 2)
            def _():
                wait(out_copy(i - 2, b))

            ob = o_v.at[b]
            gb = g_v.at[b]
            wb = w_v.at[b]

            @pl.loop(0, _CH)
            def row(r):
                acc = [[jnp.zeros((16,), f32) for _ in range(4)]
                       for _ in range(4)]
                base = (r // _H) * 512 + (r % _H) * 16
                himask = jnp.full((16,), -65536, jnp.int32)
                for c in range(4):
                    cbase = base + c * 128
                    w16 = wb[pl.ds(cbase, 16)]
                    for lp in range(16):
                        e = cbase + lp
                        gi = gb[e, pl.ds(0, 16)]
                        g0 = lax.bitcast_convert_type(
                            jnp.left_shift(gi, 16), f32)
                        g1 = lax.bitcast_convert_type(gi & himask, f32)
                        s = w16[lp]
                        p = lp % 2
                        acc[c][2 * p] = acc[c][2 * p] + g0 * s
                        acc[c][2 * p + 1] = acc[c][2 * p + 1] + g1 * s
                a0 = ((acc[0][0] + acc[0][2]) + (acc[1][0] + acc[1][2])) + \
                     ((acc[2][0] + acc[2][2]) + (acc[3][0] + acc[3][2]))
                a1 = ((acc[0][1] + acc[0][3]) + (acc[1][1] + acc[1][3])) + \
                     ((acc[2][1] + acc[2][3]) + (acc[3][1] + acc[3][3]))
                ob[pl.ds(r * _HD, 16)] = a0
                ob[pl.ds(r * _HD + 16, 16)] = a1

            start(out_copy(i, b))

            @pl.when(i + 2 < _NCH)
            def _():
                start(in_copies(i + 2, b))

        @pl.loop(0, (_NCH + 1) // 2)
        def pair(p):
            for b in (0, 1):
                i = p * 2 + b

                @pl.when(i < _NCH)
                def _():
                    step(i, b)

        wait(out_copy(_NCH - 2, (_NCH - 2) % 2))
        wait(out_copy(_NCH - 1, (_NCH - 1) % 2))

    return k(value_rows, idx_all, w_all)


def _stage_c_body(x_ref, W_ref, b_ref, o_ref):
    o_ref[...] = (jnp.dot(x_ref[...], W_ref[...],
                          preferred_element_type=jnp.float32) + b_ref[...])


def _stage_c(x2, Wout, bout2):
    grid = 4
    bq = _NQ // grid
    return pl.pallas_call(
        _stage_c_body,
        grid=(grid,),
        in_specs=[pl.BlockSpec((bq, _D), lambda i: (i, 0)),
                  pl.BlockSpec((_D, _D), lambda i: (0, 0)),
                  pl.BlockSpec((1, _D), lambda i: (0, 0))],
        out_specs=pl.BlockSpec((bq, _D), lambda i: (i, 0)),
        out_shape=jax.ShapeDtypeStruct((_NQ, _D), jnp.float32),
    )(x2, Wout, bout2)


def kernel(query, reference_points, input_flatten, input_spatial_shapes,
           input_level_start_index, Wv, bv, Woff, boff, Wattn, battn,
           Wout, bout):
    f32 = jnp.float32
    q2 = query[0]
    rp = reference_points[0]
    x2 = input_flatten[0]
    rpx = rp[..., 0]
    rpy = rp[..., 1]
    Woff6 = Woff.reshape(_D, _H, _L, _P, 2)
    Wox = Woff6[..., 0].reshape(_D, 128)
    Woy = Woff6[..., 1].reshape(_D, 128)
    boff6 = boff.reshape(_H, _L, _P, 2)
    box = boff6[..., 0].reshape(1, 128)
    boy = boff6[..., 1].reshape(1, 128)
    ba2 = battn.reshape(1, 128)
    perm = jnp.asarray(_PERM)
    Wv_p = Wv[:, perm]
    bv2 = bv[perm].reshape(1, _D)

    (value, sx, sy, aw128, idxq, wq) = _stage_a(
        q2, rpx, rpy, x2, Wv_p, bv2, Wox, Woy, box, boy, Wattn, ba2)

    value_rows = jax.lax.bitcast_convert_type(
        value.reshape(_NROWS, 16, 2), jnp.int32)
    out_flat = _sc_combine(value_rows, idxq.reshape(-1), wq.reshape(-1))

    out = _stage_c(out_flat.reshape(_NQ, _D), Wout, bout.reshape(1, _D))

    sampling_locations = jnp.stack(
        [sx.reshape(1, _NQ, _H, _L, _P), sy.reshape(1, _NQ, _H, _L, _P)],
        axis=-1)
    aw = aw128.reshape(1, _NQ, _H, _L, _P)
    return (out.reshape(1, _NQ, _D).astype(f32), sampling_locations, aw)


# speedup vs baseline: 4.4021x; 4.4021x over previous
import functools

import numpy as np
import jax
import jax.numpy as jnp
from jax import lax
from jax.experimental import pallas as pl
from jax.experimental.pallas import tpu as pltpu
from jax.experimental.pallas import tpu_sc as plsc

_D = 256
_H = 8
_L = 4
_P = 4
_HD = _D // _H
_SPATIAL = np.array([[64, 64], [32, 32], [16, 16], [8, 8]], dtype=np.int64)
_LSI = np.concatenate([np.zeros(1, np.int64),
                       np.cumsum(_SPATIAL[:, 0] * _SPATIAL[:, 1])[:-1]])
_NK = int((_SPATIAL[:, 0] * _SPATIAL[:, 1]).sum())
_NQ = _NK
_NROWS = _NQ * _H

_j = np.arange(_H * _L * _P)
_l_of_j = (_j // _P) % _L
_h_of_j = _j // (_L * _P)
_WJ = _SPATIAL[_l_of_j, 1].astype(np.float32).reshape(1, 128)
_HJ = _SPATIAL[_l_of_j, 0].astype(np.float32).reshape(1, 128)
_WJI = _SPATIAL[_l_of_j, 1].astype(np.int32).reshape(1, 128)
_HJI = _SPATIAL[_l_of_j, 0].astype(np.int32).reshape(1, 128)
_BASEJ = (_LSI[_l_of_j] * _H + _h_of_j).astype(np.int32).reshape(1, 128)
_GS = np.kron(np.eye(_H, dtype=np.float32),
              np.ones((_L * _P, _L * _P), np.float32))

_PERM_LO = np.concatenate([np.arange(16) + h * 32 for h in range(_H)])
_PERM_HI = _PERM_LO + 16

_NWORK = 32
_RPW = _NROWS // _NWORK
_CHQ = 5
_CH = _CHQ * _H
_NCH = _RPW // _CH
_EPC = _CH * 64


def _stage_a_body(q_ref, rpx_ref, rpy_ref, x_ref, Wvlo_ref, Wvhi_ref,
                  bvlo_ref, bvhi_ref,
                  Wox_ref, Woy_ref, box_ref, boy_ref, Wa_ref, ba_ref,
                  gs_ref, wjf_ref, hjf_ref, wji_ref, hji_ref, basej_ref,
                  value_ref, sx_ref, sy_ref, aw_ref, idx_ref, w_ref):
    f32 = jnp.float32
    i32 = jnp.int32
    q = q_ref[...]
    vlo = jnp.dot(x_ref[...], Wvlo_ref[...],
                  preferred_element_type=f32) + bvlo_ref[...]
    vhi = jnp.dot(x_ref[...], Wvhi_ref[...],
                  preferred_element_type=f32) + bvhi_ref[...]

    def rtne(v):
        b = lax.bitcast_convert_type(v, jnp.uint32)
        return b + jnp.uint32(0x7FFF) + ((b >> 16) & jnp.uint32(1))

    word = (rtne(vhi) & jnp.uint32(0xFFFF0000)) | (rtne(vlo) >> 16)
    value_ref[...] = lax.bitcast_convert_type(word, i32)
    offx = jnp.dot(q, Wox_ref[...], preferred_element_type=f32) + box_ref[...]
    offy = jnp.dot(q, Woy_ref[...], preferred_element_type=f32) + boy_ref[...]
    logits = jnp.dot(q, Wa_ref[...], preferred_element_type=f32) + ba_ref[...]
    e = jnp.exp(logits)
    aw = e / jnp.dot(e, gs_ref[...], preferred_element_type=f32)
    aw_ref[...] = aw

    wjf = wjf_ref[...]
    hjf = hjf_ref[...]
    lane = lax.broadcasted_iota(jnp.int32, (1, 128), 1)
    lane_l = (lane // _P) % _L
    rx = jnp.zeros_like(offx)
    ry = jnp.zeros_like(offy)
    for l in range(_L):
        m = (lane_l == l).astype(f32)
        rx = rx + rpx_ref[:, l:l + 1] * m
        ry = ry + rpy_ref[:, l:l + 1] * m
    locx = rx + offx / wjf
    locy = ry + offy / hjf
    sx_ref[...] = locx
    sy_ref[...] = locy

    x = locx * wjf - 0.5
    y = locy * hjf - 0.5
    x0 = jnp.floor(x)
    y0 = jnp.floor(y)
    lx = x - x0
    ly = y - y0
    xi = x0.astype(jnp.int32)
    yi = y0.astype(jnp.int32)
    wji = wji_ref[...]
    hji = hji_ref[...]
    basej = basej_ref[...]

    def corner(dx, dy, wgt):
        cx = xi + dx
        cy = yi + dy
        valid = (cx >= 0) & (cx < wji) & (cy >= 0) & (cy < hji)
        ccx = jnp.clip(cx, 0, wji - 1)
        ccy = jnp.clip(cy, 0, hji - 1)
        idx = basej + (ccy * wji + ccx) * _H
        w = jnp.where(valid, wgt, 0.0) * aw
        return idx, w

    bq = xi.shape[0]
    for c, (dx, dy, wgt) in enumerate((
            (0, 0, (1 - lx) * (1 - ly)), (1, 0, lx * (1 - ly)),
            (0, 1, (1 - lx) * ly), (1, 1, lx * ly))):
        idx, w = corner(dx, dy, wgt)
        idx_ref[:, c:c + 1, :] = idx.reshape(bq, 1, 128)
        w_ref[:, c:c + 1, :] = w.reshape(bq, 1, 128)


def _stage_a(q2, rpx, rpy, x2, Wvlo, Wvhi, bvlo, bvhi,
             Wox, Woy, box, boy, Wa, ba2):
    grid = 4
    bq = _NQ // grid
    f32 = jnp.float32
    i32 = jnp.int32
    row_spec = lambda w: pl.BlockSpec((bq, w), lambda i: (i, 0))
    full_spec = lambda a: pl.BlockSpec(a.shape, lambda i: (0,) * a.ndim)
    consts = [jnp.asarray(c) for c in
              (_GS, _WJ, _HJ, _WJI, _HJI, _BASEJ)]
    out_shapes = ([jax.ShapeDtypeStruct((_NQ, 128), i32)]
                  + [jax.ShapeDtypeStruct((_NQ, 128), f32)] * 3
                  + [jax.ShapeDtypeStruct((_NQ, 4, 128), i32),
                     jax.ShapeDtypeStruct((_NQ, 4, 128), f32)])
    out_specs = ([row_spec(128)] + [row_spec(128)] * 3
                 + [pl.BlockSpec((bq, 4, 128), lambda i: (i, 0, 0))] * 2)
    in_arrays = (q2, rpx, rpy, x2, Wvlo, Wvhi, bvlo, bvhi,
                 Wox, Woy, box, boy, Wa, ba2, *consts)
    in_specs = [row_spec(_D), row_spec(_L), row_spec(_L), row_spec(_D)] + \
               [full_spec(a) for a in in_arrays[4:]]
    return pl.pallas_call(
        _stage_a_body,
        grid=(grid,),
        in_specs=in_specs,
        out_specs=out_specs,
        out_shape=out_shapes,
    )(*in_arrays)


def _sc_combine(value_rows, idx_all, w_all):
    mesh = plsc.VectorSubcoreMesh(core_axis_name="c", subcore_axis_name="s")
    f32 = jnp.float32

    @functools.partial(
        pl.kernel,
        mesh=mesh,
        compiler_params=pltpu.CompilerParams(use_tc_tiling_on_sc=False),
        out_type=jax.ShapeDtypeStruct((_NROWS * _HD,), f32),
        scratch_types=[
            pltpu.VMEM((2, _EPC), jnp.int32),
            pltpu.VMEM((2, _EPC), f32),
            pltpu.VMEM((2, _EPC, 16), jnp.int32),
            pltpu.VMEM((2, _CH * _HD), f32),
            pltpu.SemaphoreType.DMA,
            pltpu.SemaphoreType.DMA,
            pltpu.SemaphoreType.DMA,
            pltpu.SemaphoreType.DMA,
            pltpu.SemaphoreType.DMA,
            pltpu.SemaphoreType.DMA,
        ],
    )
    def k(val_hbm, idx_hbm, w_hbm, out_hbm,
          idx_v, w_v, g_v, o_v,
          sem_in0, sem_in1, sem_g0, sem_g1, sem_o0, sem_o1):
        sem_in = (sem_in0, sem_in1)
        sem_g = (sem_g0, sem_g1)
        sem_o = (sem_o0, sem_o1)
        wid = lax.axis_index("s") * 2 + lax.axis_index("c")
        row0 = wid * _RPW

        def in_copies(ci, b):
            e0 = (row0 + ci * _CH) * 64
            return (pltpu.make_async_copy(idx_hbm.at[pl.ds(e0, _EPC)],
                                          idx_v.at[b], sem_in[b]),
                    pltpu.make_async_copy(w_hbm.at[pl.ds(e0, _EPC)],
                                          w_v.at[b], sem_in[b]))

        def gather(b):
            return pltpu.make_async_copy(val_hbm.at[idx_v.at[b]],
                                         g_v.at[b], sem_g[b])

        def out_copy(ci, b):
            return pltpu.make_async_copy(
                o_v.at[b],
                out_hbm.at[pl.ds((row0 + ci * _CH) * _HD, _CH * _HD)],
                sem_o[b])

        def start(copies):
            for cp in (copies if isinstance(copies, tuple) else (copies,)):
                cp.start()

        def wait(copies):
            for cp in (copies if isinstance(copies, tuple) else (copies,)):
                cp.wait()

        start(in_copies(0, 0))
        start(in_copies(1, 1))
        wait(in_copies(0, 0))
        start(gather(0))

        def step(i, b):
            wait(gather(b))

            @pl.when(i + 1 < _NCH)
            def _():
                wait(in_copies(i + 1, 1 - b))
                start(gather(1 - b))

            @pl.when(i >= 2)
            def _():
                wait(out_copy(i - 2, b))

            ob = o_v.at[b]
            gb = g_v.at[b]
            wb = w_v.at[b]

            @pl.loop(0, _CH)
            def row(r):
                acc = [[jnp.zeros((16,), f32) for _ in range(4)]
                       for _ in range(4)]
                base = (r // _H) * 512 + (r % _H) * 16
                himask = jnp.full((16,), -65536, jnp.int32)
                for c in range(4):
                    cbase = base + c * 128
                    w16 = wb[pl.ds(cbase, 16)]
                    for lp in range(16):
                        e = cbase + lp
                        gi = gb[e, pl.ds(0, 16)]
                        g0 = lax.bitcast_convert_type(
                            jnp.left_shift(gi, 16), f32)
                        g1 = lax.bitcast_convert_type(gi & himask, f32)
                        s = w16[lp]
                        p = lp % 2
                        acc[c][2 * p] = acc[c][2 * p] + g0 * s
                        acc[c][2 * p + 1] = acc[c][2 * p + 1] + g1 * s
                a0 = ((acc[0][0] + acc[0][2]) + (acc[1][0] + acc[1][2])) + \
                     ((acc[2][0] + acc[2][2]) + (acc[3][0] + acc[3][2]))
                a1 = ((acc[0][1] + acc[0][3]) + (acc[1][1] + acc[1][3])) + \
                     ((acc[2][1] + acc[2][3]) + (acc[3][1] + acc[3][3]))
                ob[pl.ds(r * _HD, 16)] = a0
                ob[pl.ds(r * _HD + 16, 16)] = a1

            start(out_copy(i, b))

            @pl.when(i + 2 < _NCH)
            def _():
                start(in_copies(i + 2, b))

        @pl.loop(0, (_NCH + 1) // 2)
        def pair(p):
            for b in (0, 1):
                i = p * 2 + b

                @pl.when(i < _NCH)
                def _():
                    step(i, b)

        wait(out_copy(_NCH - 2, (_NCH - 2) % 2))
        wait(out_copy(_NCH - 1, (_NCH - 1) % 2))

    return k(value_rows, idx_all, w_all)


def _stage_c_body(x_ref, W_ref, b_ref, o_ref):
    o_ref[...] = (jnp.dot(x_ref[...], W_ref[...],
                          preferred_element_type=jnp.float32) + b_ref[...])


def _stage_c(x2, Wout, bout2):
    grid = 4
    bq = _NQ // grid
    return pl.pallas_call(
        _stage_c_body,
        grid=(grid,),
        in_specs=[pl.BlockSpec((bq, _D), lambda i: (i, 0)),
                  pl.BlockSpec((_D, _D), lambda i: (0, 0)),
                  pl.BlockSpec((1, _D), lambda i: (0, 0))],
        out_specs=pl.BlockSpec((bq, _D), lambda i: (i, 0)),
        out_shape=jax.ShapeDtypeStruct((_NQ, _D), jnp.float32),
    )(x2, Wout, bout2)


def kernel(query, reference_points, input_flatten, input_spatial_shapes,
           input_level_start_index, Wv, bv, Woff, boff, Wattn, battn,
           Wout, bout):
    f32 = jnp.float32
    q2 = query[0]
    rp = reference_points[0]
    x2 = input_flatten[0]
    rpx = rp[..., 0]
    rpy = rp[..., 1]
    Woff6 = Woff.reshape(_D, _H, _L, _P, 2)
    Wox = Woff6[..., 0].reshape(_D, 128)
    Woy = Woff6[..., 1].reshape(_D, 128)
    boff6 = boff.reshape(_H, _L, _P, 2)
    box = boff6[..., 0].reshape(1, 128)
    boy = boff6[..., 1].reshape(1, 128)
    ba2 = battn.reshape(1, 128)
    plo = jnp.asarray(_PERM_LO)
    phi = jnp.asarray(_PERM_HI)
    Wvlo = Wv[:, plo]
    Wvhi = Wv[:, phi]
    bvlo = bv[plo].reshape(1, 128)
    bvhi = bv[phi].reshape(1, 128)

    (vword, sx, sy, aw128, idxq, wq) = _stage_a(
        q2, rpx, rpy, x2, Wvlo, Wvhi, bvlo, bvhi,
        Wox, Woy, box, boy, Wattn, ba2)

    value_rows = vword.reshape(_NROWS, 16)
    out_flat = _sc_combine(value_rows, idxq.reshape(-1), wq.reshape(-1))

    out = _stage_c(out_flat.reshape(_NQ, _D), Wout, bout.reshape(1, _D))

    sampling_locations = jnp.stack(
        [sx.reshape(1, _NQ, _H, _L, _P), sy.reshape(1, _NQ, _H, _L, _P)],
        axis=-1)
    aw = aw128.reshape(1, _NQ, _H, _L, _P)
    return (out.reshape(1, _NQ, _D).astype(f32), sampling_locations, aw)
